# baseline (speedup 1.0000x reference)
import functools

import jax
import jax.numpy as jnp
from jax import lax
from jax.experimental import pallas as pl
from jax.experimental.pallas import tpu as pltpu
from jax.experimental.pallas import tpu_sc as plsc

N = 10000
D = 128
E = 320000
CHUNK = 128
NC, NS = 2, 16
NW = NC * NS
CW = 80
EPAD = NW * CW * CHUNK
NROWS = NW * CW + 4
NPAD = 10112
RPS = NPAD // NS
CNTW = 16
ROWS_BLK = 2000


@functools.cache
def _make_sc_agg(first: bool):
  mesh = plsc.VectorSubcoreMesh(
      core_axis_name="core", subcore_axis_name="subcore",
      num_cores=NC, num_subcores=NS)
  out_type = [jax.ShapeDtypeStruct((NC, NPAD, D), jnp.float32)]
  scratch = [
      pltpu.VMEM_SHARED((NPAD, D), jnp.float32),
      pltpu.VMEM((CHUNK, D), jnp.float32),
      pltpu.VMEM((CHUNK, D), jnp.float32),
      pltpu.VMEM((2, CHUNK), jnp.int32),
      pltpu.VMEM((2, CHUNK), jnp.int32),
      pltpu.VMEM((2, CHUNK), jnp.int32),
      pltpu.VMEM((2, CHUNK), jnp.int32),
      pltpu.SemaphoreType.DMA,
      pltpu.SemaphoreType.DMA,
      pltpu.SemaphoreType.DMA,
      pltpu.SemaphoreType.DMA,
      pltpu.SemaphoreType.DMA,
      pltpu.SemaphoreType.DMA,
      pltpu.SemaphoreType.DMA,
      pltpu.SemaphoreType.DMA,
  ]
  if first:
    out_type.append(jax.ShapeDtypeStruct((NC, NPAD, CNTW), jnp.float32))
    scratch += [
        pltpu.VMEM_SHARED((NPAD, CNTW), jnp.float32),
        pltpu.VMEM((CHUNK, CNTW), jnp.float32),
        pltpu.SemaphoreType.DMA,
        pltpu.SemaphoreType.DMA,
        pltpu.SemaphoreType.DMA,
        pltpu.SemaphoreType.DMA,
    ]

  def body(h_hbm, src_hbm, dst_hbm, z128_hbm, *rest):
    if first:
      (z16_hbm, ones_hbm, agg_out, cnt_out, agg_sh, buf0, buf1,
       ib0, ib1, ib2, ib3, gs0, gs1, ss0, ss1, is0, is1, is2, is3,
       cnt_sh, ones_v, cs0, cs1, cs2, cs3) = rest
      cnt_sems = [cs0, cs1, cs2, cs3]
    else:
      (agg_out, agg_sh, buf0, buf1, ib0, ib1, ib2, ib3,
       gs0, gs1, ss0, ss1, is0, is1, is2, is3) = rest
    bufs, gsems, ssems = [buf0, buf1], [gs0, gs1], [ss0, ss1]
    ibufs, isems = [ib0, ib1, ib2, ib3], [is0, is1, is2, is3]
    c = lax.axis_index("core")
    s = lax.axis_index("subcore")
    w = c * NS + s
    r0 = s * RPS

    def idx_load(k, chunk):
      row = w * CW + chunk
      pltpu.async_copy(src_hbm.at[row], ibufs[k].at[0], isems[k])
      pltpu.async_copy(dst_hbm.at[row], ibufs[k].at[1], isems[k])

    def idx_wait(k, chunk):
      row = w * CW + chunk
      pltpu.make_async_copy(src_hbm.at[row], ibufs[k].at[0], isems[k]).wait()
      pltpu.make_async_copy(dst_hbm.at[row], ibufs[k].at[1], isems[k]).wait()

    def gather(b, k):
      pltpu.async_copy(h_hbm.at[ibufs[k].at[0]], bufs[b], gsems[b])

    def gather_wait(b, k):
      pltpu.make_async_copy(h_hbm.at[ibufs[k].at[0]], bufs[b],
                            gsems[b]).wait()

    def scat(b, k):
      pltpu.async_copy(bufs[b], agg_sh.at[ibufs[k].at[1]], ssems[b],
                       add=True)
      if first:
        pltpu.async_copy(ones_v, cnt_sh.at[ibufs[k].at[1]], cnt_sems[k],
                         add=True)

    def scat_wait(b, k):
      pltpu.make_async_copy(bufs[b], agg_sh.at[ibufs[k].at[1]],
                            ssems[b]).wait()

    def cnt_wait(k):
      if first:
        pltpu.make_async_copy(ones_v, cnt_sh.at[ibufs[k].at[1]],
                              cnt_sems[k]).wait()

    pltpu.sync_copy(z128_hbm.at[pl.ds(r0, RPS)], agg_sh.at[pl.ds(r0, RPS)])
    if first:
      pltpu.sync_copy(z16_hbm.at[pl.ds(r0, RPS)], cnt_sh.at[pl.ds(r0, RPS)])
      pltpu.sync_copy(ones_hbm, ones_v)
    plsc.subcore_barrier()

    for k in range(4):
      idx_load(k, k)
    idx_wait(0, 0)
    gather(0, 0)
    idx_wait(1, 1)
    gather(1, 1)

    @pl.loop(0, CW, step=4)
    def _(i):
      gather_wait(0, 0)
      scat(0, 0)
      gather_wait(1, 1)
      scat(1, 1)
      scat_wait(0, 0)
      cnt_wait(0)
      idx_load(0, i + 4)
      idx_wait(2, i + 2)
      gather(0, 2)
      scat_wait(1, 1)
      cnt_wait(1)
      idx_load(1, i + 5)
      idx_wait(3, i + 3)
      gather(1, 3)
      gather_wait(0, 2)
      scat(0, 2)
      gather_wait(1, 3)
      scat(1, 3)
      scat_wait(0, 2)
      cnt_wait(2)
      idx_load(2, i + 6)
      idx_wait(0, i + 4)
      gather(0, 0)
      scat_wait(1, 3)
      cnt_wait(3)
      idx_load(3, i + 7)
      idx_wait(1, i + 5)
      gather(1, 1)

    gather_wait(0, 0)
    gather_wait(1, 1)
    idx_wait(2, CW + 2)
    idx_wait(3, CW + 3)

    plsc.subcore_barrier()
    pltpu.sync_copy(agg_sh.at[pl.ds(r0, RPS)], agg_out.at[c, pl.ds(r0, RPS)])
    if first:
      pltpu.sync_copy(cnt_sh.at[pl.ds(r0, RPS)], cnt_out.at[c, pl.ds(r0, RPS)])

  return pl.kernel(body, out_type=tuple(out_type) if first else out_type[0],
                   mesh=mesh, scratch_types=scratch,
                   compiler_params=pltpu.CompilerParams(
                       use_tc_tiling_on_sc=False))


def _elu(z):
  return jnp.where(z > 0, z, jnp.exp(z) - 1.0)


def _tc_first_body(agg0, agg1, cnt0, cnt1, h, wl, wr, b, out, invout):
  cnt = cnt0[0][:, 0:1] + cnt1[0][:, 0:1]
  inv = 1.0 / jnp.maximum(cnt, 1.0)
  invout[...] = jnp.broadcast_to(inv, (ROWS_BLK, D))
  mean = (agg0[0] + agg1[0]) * inv
  z = (jnp.dot(mean, wl[...], preferred_element_type=jnp.float32,
               precision=lax.Precision.HIGHEST)
       + jnp.dot(h[...], wr[...], preferred_element_type=jnp.float32,
                 precision=lax.Precision.HIGHEST) + b[...])
  out[...] = _elu(z)


def _tc_mid_body(last, agg0, agg1, inv, h, wl, wr, b, out):
  mean = (agg0[0] + agg1[0]) * inv[...]
  z = (jnp.dot(mean, wl[...], preferred_element_type=jnp.float32,
               precision=lax.Precision.HIGHEST)
       + jnp.dot(h[...], wr[...], preferred_element_type=jnp.float32,
                 precision=lax.Precision.HIGHEST) + b[...])
  if last:
    m = jnp.max(z, axis=-1, keepdims=True)
    lse = jnp.log(jnp.sum(jnp.exp(z - m), axis=-1, keepdims=True)) + m
    out[...] = z - lse
  else:
    out[...] = _elu(z)


_agg_spec = lambda core: pl.BlockSpec((1, ROWS_BLK, D), lambda i, c=core: (c, i, 0))
_cnt_spec = lambda core: pl.BlockSpec((1, ROWS_BLK, CNTW), lambda i, c=core: (c, i, 0))
_row_spec = pl.BlockSpec((ROWS_BLK, D), lambda i: (i, 0))
_w_spec = pl.BlockSpec((D, D), lambda i: (0, 0))
_b_spec = pl.BlockSpec((1, D), lambda i: (0, 0))
_GRID = (N // ROWS_BLK,)

_tc_first = pl.pallas_call(
    _tc_first_body,
    grid=_GRID,
    in_specs=[_agg_spec(0), _agg_spec(1), _cnt_spec(0), _cnt_spec(1),
              _row_spec, _w_spec, _w_spec, _b_spec],
    out_specs=[_row_spec, _row_spec],
    out_shape=[jax.ShapeDtypeStruct((N, D), jnp.float32),
               jax.ShapeDtypeStruct((N, D), jnp.float32)],
)

_tc_mid = pl.pallas_call(
    functools.partial(_tc_mid_body, False),
    grid=_GRID,
    in_specs=[_agg_spec(0), _agg_spec(1), _row_spec,
              _row_spec, _w_spec, _w_spec, _b_spec],
    out_specs=_row_spec,
    out_shape=jax.ShapeDtypeStruct((N, D), jnp.float32),
)

_tc_last = pl.pallas_call(
    functools.partial(_tc_mid_body, True),
    grid=_GRID,
    in_specs=[_agg_spec(0), _agg_spec(1), _row_spec,
              _row_spec, _w_spec, _w_spec, _b_spec],
    out_specs=_row_spec,
    out_shape=jax.ShapeDtypeStruct((N, D), jnp.float32),
)


def kernel(x, edge_index, W1l, W1r, b1, W2l, W2r, b2, W3l, W3r, b3,
           W4l, W4r, b4):
  src = edge_index[0].astype(jnp.int32)
  dst = edge_index[1].astype(jnp.int32)
  npad_e = NROWS * CHUNK - E
  src_p = jnp.concatenate(
      [src, jnp.zeros((npad_e,), jnp.int32)]).reshape(NROWS, CHUNK)
  dst_p = jnp.concatenate(
      [dst, N + (jnp.arange(npad_e, dtype=jnp.int32) % (NPAD - N))]
  ).reshape(NROWS, CHUNK)
  z128 = jnp.zeros((NPAD, D), jnp.float32)
  z16 = jnp.zeros((NPAD, CNTW), jnp.float32)
  ones16 = jnp.ones((CHUNK, CNTW), jnp.float32)

  agg1, cnt = _make_sc_agg(True)(x, src_p, dst_p, z128, z16, ones16)
  _sc_agg = _make_sc_agg(False)
  h1, inv = _tc_first(agg1, agg1, cnt, cnt, x,
                      W1l, W1r, b1.reshape(1, D))
  agg2 = _sc_agg(h1, src_p, dst_p, z128)
  h2 = _tc_mid(agg2, agg2, inv, h1, W2l, W2r, b2.reshape(1, D))
  agg3 = _sc_agg(h2, src_p, dst_p, z128)
  h3 = _tc_mid(agg3, agg3, inv, h2, W3l, W3r, b3.reshape(1, D))
  agg4 = _sc_agg(h3, src_p, dst_p, z128)

  dout = W4l.shape[1]
  W4l_p = jnp.zeros((D, D), jnp.float32).at[:, :dout].set(W4l)
  W4r_p = jnp.zeros((D, D), jnp.float32).at[:, :dout].set(W4r)
  b4_p = jnp.full((1, D), -1e30, jnp.float32).at[0, :dout].set(b4)
  out = _tc_last(agg4, agg4, inv, h3, W4l_p, W4r_p, b4_p)
  return out[:, :dout]
